# baseline (speedup 1.0000x reference)
import jax
import jax.numpy as jnp
from jax.experimental import pallas as pl
from jax.experimental.pallas import tpu as pltpu

N = 2048
M = 2048
TM = 512
NT = M // TM


def _chamfer_body(x_ref, yt_ref, o_ref):
    nb = x_ref.shape[0]

    def batch_step(b, tot):
        x0 = x_ref[b, :, 0:1]
        x1 = x_ref[b, :, 1:2]
        x2 = x_ref[b, :, 2:3]
        m1 = None
        s2 = None
        for t in range(NT):
            sl = pl.ds(t * TM, TM)
            d0 = x0 - yt_ref[b, 0:1, sl]
            acc = d0 * d0
            d1 = x1 - yt_ref[b, 1:2, sl]
            acc = acc + d1 * d1
            d2 = x2 - yt_ref[b, 2:3, sl]
            acc = acc + d2 * d2
            m1t = jnp.min(acc, axis=1, keepdims=True)
            m1 = m1t if m1 is None else jnp.minimum(m1, m1t)
            s2t = jnp.sum(jnp.min(acc, axis=0, keepdims=True).astype(jnp.float32))
            s2 = s2t if s2 is None else s2 + s2t
        s1 = jnp.sum(m1.astype(jnp.float32))
        return tot + s1 + s2

    tot = jax.lax.fori_loop(0, nb, batch_step, jnp.float32(0.0))
    o_ref[...] = jnp.full((1, 1), tot * (1.0 / (N * nb)), jnp.float32)


@jax.jit
def kernel(receptive_pc, decoder_pc):
    b = receptive_pc.shape[0]
    xb = receptive_pc.astype(jnp.bfloat16)
    yt = jnp.swapaxes(decoder_pc, 1, 2).astype(jnp.bfloat16)
    out = pl.pallas_call(
        _chamfer_body,
        out_shape=jax.ShapeDtypeStruct((1, 1), jnp.float32),
    )(xb, yt)
    return out.reshape(())


# speedup vs baseline: 1.3118x; 1.3118x over previous
import jax
import jax.numpy as jnp
from jax.experimental import pallas as pl
from jax.experimental.pallas import tpu as pltpu

N = 2048
M = 2048
TM = 512
NT = M // TM


def _chamfer_body(xs_ref, yt_ref, o_ref):
    b = pl.program_id(0)
    nb = pl.num_programs(0)

    xf = xs_ref[...].astype(jnp.float32) * -0.5
    x2 = jnp.sum(xf * xf, axis=1, keepdims=True)
    yf = yt_ref[...].astype(jnp.float32)
    y2 = jnp.sum(yf * yf, axis=0, keepdims=True)

    m1 = None
    s2acc = None
    for t in range(NT):
        sl = pl.ds(t * TM, TM)
        w = jax.lax.dot_general(
            xs_ref[...],
            yt_ref[:, sl],
            (((1,), (0,)), ((), ())),
            preferred_element_type=jnp.float32,
        )
        ttile = w + y2[0:1, t * TM : (t + 1) * TM]
        m1t = jnp.min(ttile, axis=1, keepdims=True)
        m1 = m1t if m1 is None else jnp.minimum(m1, m1t)
        ct = jnp.min(ttile + x2, axis=0, keepdims=True)
        s2acc = ct if s2acc is None else s2acc + ct
    s1 = jnp.sum(m1 + x2)
    s2 = jnp.sum(s2acc)
    step = (s1 + s2) * (1.0 / (N * nb))

    @pl.when(b == 0)
    def _init():
        o_ref[...] = jnp.zeros_like(o_ref)

    o_ref[...] += step


@jax.jit
def kernel(receptive_pc, decoder_pc):
    b = receptive_pc.shape[0]
    xs = (-2.0 * receptive_pc).astype(jnp.bfloat16)
    yt = jnp.swapaxes(decoder_pc, 1, 2).astype(jnp.bfloat16)
    out = pl.pallas_call(
        _chamfer_body,
        grid=(b,),
        in_specs=[
            pl.BlockSpec((None, N, 3), lambda i: (i, 0, 0)),
            pl.BlockSpec((None, 3, M), lambda i: (i, 0, 0)),
        ],
        out_specs=pl.BlockSpec((1, 1), lambda i: (0, 0)),
        out_shape=jax.ShapeDtypeStruct((1, 1), jnp.float32),
        compiler_params=pltpu.CompilerParams(
            dimension_semantics=("arbitrary",),
        ),
    )(xs, yt)
    return out.reshape(())


# speedup vs baseline: 1.3807x; 1.0525x over previous
import jax
import jax.numpy as jnp
from jax.experimental import pallas as pl
from jax.experimental.pallas import tpu as pltpu

N = 2048
M = 2048
K = 6
TM = 512
NT = M // TM


def _chamfer_body(xf_ref, yf_ref, y2_ref, o_ref):
    b = pl.program_id(0)
    nb = pl.num_programs(0)
    m1 = None
    s2acc = None
    for t in range(NT):
        w = jax.lax.dot_general(
            xf_ref[...],
            yf_ref[:, pl.ds(t * TM, TM)],
            (((1,), (0,)), ((), ())),
            preferred_element_type=jnp.float32,
        )
        dist = w + y2_ref[0:1, t * TM : (t + 1) * TM]
        m1t = jnp.min(dist, axis=1, keepdims=True)
        m1 = m1t if m1 is None else jnp.minimum(m1, m1t)
        ct = jnp.min(dist, axis=0, keepdims=True)
        s2acc = ct if s2acc is None else s2acc + ct
    s1 = jnp.sum(m1)
    s2 = jnp.sum(s2acc)
    step = (s1 + s2) * (1.0 / (N * nb))

    @pl.when(b == 0)
    def _init():
        o_ref[...] = jnp.zeros_like(o_ref)

    o_ref[...] += step


def _split3(v):
    h = v.astype(jnp.bfloat16)
    r = v - h.astype(jnp.float32)
    m = r.astype(jnp.bfloat16)
    l = (r - m.astype(jnp.float32)).astype(jnp.bfloat16)
    return h, m, l


@jax.jit
def kernel(receptive_pc, decoder_pc):
    b = receptive_pc.shape[0]
    xb = receptive_pc.astype(jnp.bfloat16)
    yb = decoder_pc.astype(jnp.bfloat16)
    xf32 = xb.astype(jnp.float32)
    yf32 = yb.astype(jnp.float32)
    x2 = jnp.sum(xf32 * xf32, axis=2, keepdims=True)
    y2 = jnp.sum(yf32 * yf32, axis=2, keepdims=True)
    x2h, x2m, x2l = _split3(x2)
    ones_y = jnp.ones((b, M, 1), jnp.bfloat16)
    xfeat = jnp.concatenate(
        [(-2.0 * xf32).astype(jnp.bfloat16), x2h, x2m, x2l], axis=2
    )
    yfeat = jnp.concatenate([yb, ones_y, ones_y, ones_y], axis=2)
    yfeat_t = jnp.swapaxes(yfeat, 1, 2)
    y2t = jnp.swapaxes(y2, 1, 2)
    out = pl.pallas_call(
        _chamfer_body,
        grid=(b,),
        in_specs=[
            pl.BlockSpec((None, N, K), lambda i: (i, 0, 0)),
            pl.BlockSpec((None, K, M), lambda i: (i, 0, 0)),
            pl.BlockSpec((None, 1, M), lambda i: (i, 0, 0)),
        ],
        out_specs=pl.BlockSpec((1, 1), lambda i: (0, 0)),
        out_shape=jax.ShapeDtypeStruct((1, 1), jnp.float32),
        compiler_params=pltpu.CompilerParams(
            dimension_semantics=("arbitrary",),
        ),
    )(xfeat, yfeat_t, y2t)
    return out.reshape(())
